# baseline (speedup 1.0000x reference)
import functools
import math

import jax
import jax.numpy as jnp
from jax import lax
from jax.experimental import pallas as pl
from jax.experimental.pallas import tpu as pltpu
from jax.experimental.pallas import tpu_sc as plsc

EMBED = 64
SCALE = 8.0
NC = 2
NS = 16
NW = NC * NS
LANES = 16
PACK = EMBED // 2
TB = 128


def _make_gather(batch, hist):
    mesh = plsc.VectorSubcoreMesh(core_axis_name="c", subcore_axis_name="s")
    nbt = batch // TB
    bt_per_w = nbt // NW
    rows_w = bt_per_w * TB
    units = bt_per_w * hist

    @functools.partial(
        pl.kernel,
        mesh=mesh,
        compiler_params=pltpu.CompilerParams(use_tc_tiling_on_sc=False),
        out_type=jax.ShapeDtypeStruct((hist, 8, nbt, 8, TB), jnp.int32),
        scratch_types=[
            pltpu.VMEM((hist, rows_w), jnp.int32),
            pltpu.VMEM((TB, PACK), jnp.int32),
            pltpu.VMEM((TB, PACK), jnp.int32),
            pltpu.VMEM((8, 8, TB), jnp.int32),
            pltpu.VMEM((8, 8, TB), jnp.int32),
            pltpu.SemaphoreType.DMA,
            pltpu.SemaphoreType.DMA,
            pltpu.SemaphoreType.DMA,
            pltpu.SemaphoreType.DMA,
        ],
    )
    def gather_kernel(idx_hbm, table_hbm, out_hbm, idx_v,
                      gb0, gb1, tb0, tb1, gs0, gs1, os0, os1):
        wid = lax.axis_index("s") * NC + lax.axis_index("c")
        bt0 = wid * bt_per_w
        pltpu.sync_copy(idx_hbm.at[:, pl.ds(bt0 * TB, rows_w)], idx_v)

        def fire(u, gb, gs):
            bt = u // hist
            h = u % hist
            pltpu.async_copy(
                table_hbm.at[idx_v.at[h, pl.ds(bt * TB, TB)]], gb, gs
            )

        def process(u, gb, gs, tb, osem, ogb, ogs):
            fire(jnp.minimum(u + 1, units - 1), ogb, ogs)

            pltpu.make_async_copy(
                table_hbm.at[idx_v.at[0, pl.ds(0, TB)]], gb, gs
            ).wait()

            bt = u // hist
            h = u % hist

            @pl.when(u >= 2)
            def _():
                pltpu.make_async_copy(
                    tb, out_hbm.at[h, :, bt0 + bt], osem
                ).wait()

            def tc_body(tc, carry):
                t0 = tc * LANES
                iota = lax.iota(jnp.int32, LANES)
                for pc in range(PACK // LANES):
                    p0 = pc * LANES
                    v = [gb[t0 + i, pl.ds(p0, LANES)] for i in range(LANES)]
                    for d in (1, 2, 4, 8):
                        mask = (iota & d) == 0
                        rm = (iota + (LANES - d)) % LANES
                        rp = (iota + d) % LANES
                        for i in range(LANES):
                            if i & d:
                                continue
                            a, b = v[i], v[i + d]
                            br = b.at[rm].get(mode="promise_in_bounds")
                            ar = a.at[rp].get(mode="promise_in_bounds")
                            v[i] = jnp.where(mask, a, br)
                            v[i + d] = jnp.where(mask, ar, b)
                    for j in range(LANES):
                        p = p0 + j
                        lo = v[j] << 16
                        hi = v[j] & jnp.int32(-65536)
                        e0 = 2 * p
                        e1 = 2 * p + 1
                        tb[e0 // 8, e0 % 8, pl.ds(t0, LANES)] = lo
                        tb[e1 // 8, e1 % 8, pl.ds(t0, LANES)] = hi
                return carry

            lax.fori_loop(0, TB // LANES, tc_body, 0)
            pltpu.async_copy(tb, out_hbm.at[h, :, bt0 + bt], osem)

        fire(0, gb0, gs0)

        def pair_body(g, carry):
            process(2 * g, gb0, gs0, tb0, os0, gb1, gs1)
            process(2 * g + 1, gb1, gs1, tb1, os1, gb0, gs0)
            return carry

        lax.fori_loop(0, units // 2, pair_body, 0)

        pltpu.make_async_copy(
            table_hbm.at[idx_v.at[0, pl.ds(0, TB)]], gb0, gs0
        ).wait()
        for tb, osem, u in ((tb0, os0, units - 2), (tb1, os1, units - 1)):
            bt = u // hist
            h = u % hist
            pltpu.make_async_copy(tb, out_hbm.at[h, :, bt0 + bt], osem).wait()

    return gather_kernel


def kernel(input_token, table):
    batch, hist = input_token.shape
    vocab = table.shape[0]
    nbt = batch // TB
    d = lax.bitcast_convert_type(
        (table * SCALE).astype(jnp.bfloat16).reshape(vocab, PACK, 2),
        jnp.int32,
    )
    z = jnp.zeros((vocab, PACK), jnp.int32)
    packed = jnp.stack([d, z], axis=1).reshape(vocab // 2, 4 * PACK)
    table_i = packed.reshape(2 * vocab, PACK)
    idxt = (input_token.astype(jnp.int32) * 2).T
    out5d = _make_gather(batch, hist)(idxt, table_i)
    outf = lax.bitcast_convert_type(out5d, jnp.float32)
    return outf.transpose(2, 4, 0, 1, 3).reshape(batch, hist, EMBED)


# speedup vs baseline: 3.5135x; 3.5135x over previous
import functools
import math

import jax
import jax.numpy as jnp
from jax import lax
from jax.experimental import pallas as pl
from jax.experimental.pallas import tpu as pltpu
from jax.experimental.pallas import tpu_sc as plsc

EMBED = 64
SCALE = 8.0
NC = 2
NS = 16
NW = NC * NS
LANES = 16
TB = 128


def _make_gather(batch, hist):
    mesh = plsc.VectorSubcoreMesh(core_axis_name="c", subcore_axis_name="s")
    nbt = batch // TB
    bt_per_w = nbt // NW
    rows_w = bt_per_w * TB
    units = bt_per_w * hist

    @functools.partial(
        pl.kernel,
        mesh=mesh,
        compiler_params=pltpu.CompilerParams(use_tc_tiling_on_sc=False),
        out_type=jax.ShapeDtypeStruct((hist, 8, nbt, 8, TB), jnp.float32),
        scratch_types=[
            pltpu.VMEM((hist, rows_w), jnp.int32),
            pltpu.VMEM((TB, EMBED), jnp.float32),
            pltpu.VMEM((TB, EMBED), jnp.float32),
            pltpu.VMEM((8, 8, TB), jnp.float32),
            pltpu.VMEM((8, 8, TB), jnp.float32),
            pltpu.SemaphoreType.DMA,
            pltpu.SemaphoreType.DMA,
            pltpu.SemaphoreType.DMA,
            pltpu.SemaphoreType.DMA,
        ],
    )
    def gather_kernel(idx_hbm, table_hbm, out_hbm, idx_v,
                      gb0, gb1, tb0, tb1, gs0, gs1, os0, os1):
        wid = lax.axis_index("s") * NC + lax.axis_index("c")
        bt0 = wid * bt_per_w
        pltpu.sync_copy(idx_hbm.at[:, pl.ds(bt0 * TB, rows_w)], idx_v)

        def fire(u, gb, gs):
            bt = u // hist
            h = u % hist
            pltpu.async_copy(
                table_hbm.at[idx_v.at[h, pl.ds(bt * TB, TB)]], gb, gs
            )

        def process(u, gb, gs, tb, osem, ogb, ogs):
            fire(jnp.minimum(u + 1, units - 1), ogb, ogs)

            pltpu.make_async_copy(
                table_hbm.at[idx_v.at[0, pl.ds(0, TB)]], gb, gs
            ).wait()

            bt = u // hist
            h = u % hist

            @pl.when(u >= 2)
            def _():
                pltpu.make_async_copy(
                    tb, out_hbm.at[h, :, bt0 + bt], osem
                ).wait()

            def tc_body(tc, carry):
                t0 = tc * LANES
                iota = lax.iota(jnp.int32, LANES)
                for ec in range(EMBED // LANES):
                    e0 = ec * LANES
                    v = [gb[t0 + i, pl.ds(e0, LANES)] * SCALE
                         for i in range(LANES)]
                    for d in (1, 2, 4, 8):
                        mask = (iota & d) == 0
                        rm = (iota + (LANES - d)) % LANES
                        rp = (iota + d) % LANES
                        for i in range(LANES):
                            if i & d:
                                continue
                            a, b = v[i], v[i + d]
                            br = b.at[rm].get(mode="promise_in_bounds")
                            ar = a.at[rp].get(mode="promise_in_bounds")
                            v[i] = jnp.where(mask, a, br)
                            v[i + d] = jnp.where(mask, ar, b)
                    for j in range(LANES):
                        e = e0 + j
                        tb[e // 8, e % 8, pl.ds(t0, LANES)] = v[j]
                return carry

            lax.fori_loop(0, TB // LANES, tc_body, 0)
            pltpu.async_copy(tb, out_hbm.at[h, :, bt0 + bt], osem)

        fire(0, gb0, gs0)

        def pair_body(g, carry):
            process(2 * g, gb0, gs0, tb0, os0, gb1, gs1)
            process(2 * g + 1, gb1, gs1, tb1, os1, gb0, gs0)
            return carry

        lax.fori_loop(0, units // 2, pair_body, 0)

        pltpu.make_async_copy(
            table_hbm.at[idx_v.at[0, pl.ds(0, TB)]], gb0, gs0
        ).wait()
        for tb, osem, u in ((tb0, os0, units - 2), (tb1, os1, units - 1)):
            bt = u // hist
            h = u % hist
            pltpu.make_async_copy(tb, out_hbm.at[h, :, bt0 + bt], osem).wait()

    return gather_kernel


def kernel(input_token, table):
    batch, hist = input_token.shape
    vocab = table.shape[0]
    nbt = batch // TB
    table_p = jnp.pad(table, ((0, 0), (0, 128 - EMBED))).reshape(2 * vocab, EMBED)
    idx2t = (input_token.astype(jnp.int32) * 2).T
    out5d = _make_gather(batch, hist)(idx2t, table_p)
    return out5d.transpose(2, 4, 0, 1, 3).reshape(batch, hist, EMBED)
